# Initial kernel scaffold; baseline (speedup 1.0000x reference)
import jax
import jax.numpy as jnp
from jax.experimental import pallas as pl


def kernel(x, edge_index, edge_weight, labels, mask, W1, W2):
    raise NotImplementedError("write your pallas kernel here")


# speedup vs baseline: 6.8546x; 6.8546x over previous
import functools

import jax
import jax.numpy as jnp
from jax import lax
from jax.experimental import pallas as pl
from jax.experimental.pallas import tpu as pltpu
from jax.experimental.pallas import tpu_sc as plsc

N = 10000
E = 320000
D = 128
H = 64
C = 2
CP = 16
WEIGHT_DECAY = 0.0005

NC = 2
NS = 16
NT = NC * NS
G = 80
NCH = E // (NT * G)
NPAD = 10240
ROWS_PER_TILE = NPAD // NS


def _sc_aggregate(table, src2d, dst2d, ew2d, F):
    mesh = plsc.VectorSubcoreMesh(core_axis_name="c", subcore_axis_name="s")
    qf = F // 16

    @functools.partial(
        pl.kernel,
        out_type=jax.ShapeDtypeStruct((NC, NPAD, F), jnp.float32),
        mesh=mesh,
        compiler_params=pltpu.CompilerParams(use_tc_tiling_on_sc=False),
        scratch_types=[
            pltpu.VMEM((NCH, G), jnp.int32),
            pltpu.VMEM((NCH, G), jnp.int32),
            pltpu.VMEM((NCH, G), jnp.float32),
            pltpu.VMEM((G, F), jnp.float32),
            pltpu.VMEM((ROWS_PER_TILE, F), jnp.float32),
            pltpu.VMEM_SHARED((NPAD, F), jnp.float32),
            pltpu.SemaphoreType.DMA,
        ],
    )
    def agg(table_hbm, src_hbm, dst_hbm, ew_hbm, out_hbm,
            src_v, dst_v, ew_v, buf, zbuf, acc, gsem):
        c = lax.axis_index("c")
        s = lax.axis_index("s")
        tid = c * NS + s

        pltpu.sync_copy(src_hbm.at[tid], src_v)
        pltpu.sync_copy(dst_hbm.at[tid], dst_v)
        pltpu.sync_copy(ew_hbm.at[tid], ew_v)

        def zrow(r, carry):
            for q in range(qf):
                zbuf[r, pl.ds(q * 16, 16)] = jnp.zeros((16,), jnp.float32)
            return carry
        lax.fori_loop(0, ROWS_PER_TILE, zrow, 0)
        pltpu.sync_copy(zbuf, acc.at[pl.ds(s * ROWS_PER_TILE, ROWS_PER_TILE)])
        plsc.subcore_barrier()

        def chunk(jj, carry):
            pltpu.async_copy(table_hbm.at[src_v.at[jj]], buf, gsem).wait()

            def mul16(t, carry2):
                wv = ew_v[jj, pl.ds(t * 16, 16)]
                eb = t * 16
                for u in range(16):
                    w = wv[u]
                    for q in range(qf):
                        sl = pl.ds(q * 16, 16)
                        buf[eb + u, sl] = buf[eb + u, sl] * w
                return carry2
            lax.fori_loop(0, G // 16, mul16, 0)

            pltpu.sync_copy(buf, acc.at[dst_v.at[jj]], add=True)
            return carry
        lax.fori_loop(0, NCH, chunk, 0)

        plsc.subcore_barrier()
        pltpu.sync_copy(acc.at[pl.ds(s * ROWS_PER_TILE, ROWS_PER_TILE)],
                        out_hbm.at[c, pl.ds(s * ROWS_PER_TILE, ROWS_PER_TILE)])

    return agg(table, src2d, dst2d, ew2d)


def _mm1_body(x_ref, w_ref, o_ref):
    o_ref[...] = jnp.dot(x_ref[...], w_ref[...],
                         preferred_element_type=jnp.float32)


def _norm_mm2_body(p_ref, w2_ref, o_ref):
    a = p_ref[0, :N] + p_ref[1, :N]
    nrm = jnp.sqrt(jnp.sum(a * a, axis=1, keepdims=True))
    h = jnp.maximum(a / jnp.maximum(nrm, 1e-12), 0.0)
    o_ref[...] = jnp.dot(h, w2_ref[...], preferred_element_type=jnp.float32)


def _loss_body(lp_ref, lab_ref, mask_ref, w1_ref, loss_ref, acc_ref):
    l = lp_ref[0, :N, 0:2] + lp_ref[1, :N, 0:2]
    l0 = l[:, 0:1]
    l1 = l[:, 1:2]
    mx = jnp.maximum(l0, l1)
    lse = mx + jnp.log(jnp.exp(l0 - mx) + jnp.exp(l1 - mx))
    lab = lab_ref[...]
    ly = jnp.where(lab == 1, l1, l0)
    ce = lse - ly
    mk = mask_ref[...]
    m = mk / jnp.mean(mk)
    wd = (WEIGHT_DECAY * 0.5) * jnp.sum(w1_ref[...] * w1_ref[...])
    loss_ref[...] = (wd + jnp.mean(ce * m)).reshape(1, 1)
    pred = (l1 > l0).astype(jnp.int32)
    corr = (pred == lab).astype(jnp.float32)
    acc_ref[...] = jnp.mean(corr * m).reshape(1, 1)


def kernel(x, edge_index, edge_weight, labels, mask, W1, W2):
    src3d = edge_index[0].reshape(NT, NCH, G)
    dst3d = edge_index[1].reshape(NT, NCH, G)
    ew3d = edge_weight.reshape(NT, NCH, G)
    w2p = jnp.zeros((H, CP), jnp.float32).at[:, :C].set(W2)

    pre1 = pl.pallas_call(
        _mm1_body,
        out_shape=jax.ShapeDtypeStruct((N, H), jnp.float32),
    )(x, W1)

    p1 = _sc_aggregate(pre1, src3d, dst3d, ew3d, H)

    pre2 = pl.pallas_call(
        _norm_mm2_body,
        out_shape=jax.ShapeDtypeStruct((N, CP), jnp.float32),
    )(p1, w2p)

    p2 = _sc_aggregate(pre2, src3d, dst3d, ew3d, CP)

    loss, acc = pl.pallas_call(
        _loss_body,
        out_shape=[jax.ShapeDtypeStruct((1, 1), jnp.float32),
                   jax.ShapeDtypeStruct((1, 1), jnp.float32)],
    )(p2, labels.reshape(N, 1), mask.reshape(N, 1), W1)

    return (loss[0, 0], acc[0, 0])


# speedup vs baseline: 10.7878x; 1.5738x over previous
import functools

import jax
import jax.numpy as jnp
from jax import lax
from jax.experimental import pallas as pl
from jax.experimental.pallas import tpu as pltpu
from jax.experimental.pallas import tpu_sc as plsc

N = 10000
E = 320000
D = 128
H = 64
C = 2
CP = 16
WEIGHT_DECAY = 0.0005

NC = 2
NS = 16
NT = NC * NS
G = 80
NCH = E // (NT * G)
NBUF = 3
PFD = 2
NPAD = 10240
ROWS_PER_TILE = NPAD // NS


def _sc_aggregate(table, src2d, dst2d, ew2d, F):
    mesh = plsc.VectorSubcoreMesh(core_axis_name="c", subcore_axis_name="s")
    qf = F // 16

    @functools.partial(
        pl.kernel,
        out_type=jax.ShapeDtypeStruct((NC, NPAD, F), jnp.float32),
        mesh=mesh,
        compiler_params=pltpu.CompilerParams(use_tc_tiling_on_sc=False),
        scratch_types=[
            pltpu.VMEM((NCH, G), jnp.int32),
            pltpu.VMEM((NCH, G), jnp.int32),
            pltpu.VMEM((NCH, G), jnp.float32),
            [pltpu.VMEM((G, F), jnp.float32)] * NBUF,
            pltpu.VMEM((ROWS_PER_TILE, F), jnp.float32),
            pltpu.VMEM_SHARED((NPAD, F), jnp.float32),
            [pltpu.SemaphoreType.DMA] * NBUF,
            [pltpu.SemaphoreType.DMA] * NBUF,
        ],
    )
    def agg(table_hbm, src_hbm, dst_hbm, ew_hbm, out_hbm,
            src_v, dst_v, ew_v, bufs, zbuf, acc, gsems, ssems):
        c = lax.axis_index("c")
        s = lax.axis_index("s")
        tid = c * NS + s

        pltpu.sync_copy(src_hbm.at[tid], src_v)
        pltpu.sync_copy(dst_hbm.at[tid], dst_v)
        pltpu.sync_copy(ew_hbm.at[tid], ew_v)

        def zrow(r, carry):
            for q in range(qf):
                zbuf[r, pl.ds(q * 16, 16)] = jnp.zeros((16,), jnp.float32)
            return carry
        lax.fori_loop(0, ROWS_PER_TILE, zrow, 0)
        pltpu.sync_copy(zbuf, acc.at[pl.ds(s * ROWS_PER_TILE, ROWS_PER_TILE)])
        plsc.subcore_barrier()

        def gather(jj, b):
            pltpu.async_copy(table_hbm.at[src_v.at[jj]], bufs[b], gsems[b])

        def scatter(jj, b):
            pltpu.async_copy(bufs[b], acc.at[dst_v.at[jj]], ssems[b], add=True)

        for j in range(PFD):
            gather(j, j % NBUF)

        def step(jj, b):
            pltpu.make_async_copy(table_hbm.at[src_v.at[jj]], bufs[b],
                                  gsems[b]).wait()

            def mul16(t, carry2):
                wv = ew_v[jj, pl.ds(t * 16, 16)]
                for u in range(16):
                    w = wv[u]
                    for q in range(qf):
                        sl = pl.ds(q * 16, 16)
                        bufs[b][t * 16 + u, sl] = bufs[b][t * 16 + u, sl] * w
                return carry2
            lax.fori_loop(0, G // 16, mul16, 0, unroll=False)

            scatter(jj, b)
            bn = (b + PFD) % NBUF

            @pl.when(jj >= NBUF - PFD)
            def _():
                pltpu.make_async_copy(bufs[bn], acc.at[dst_v.at[jj]],
                                      ssems[bn]).wait()

            @pl.when(jj + PFD < NCH)
            def _():
                gather(jj + PFD, bn)

        def block(it, carry):
            for b in range(NBUF):
                step(it * NBUF + b, b)
            return carry
        lax.fori_loop(0, NCH // NBUF, block, 0)
        for j in range(NCH - NCH % NBUF, NCH):
            step(jnp.int32(j), j % NBUF)

        for j in range(NCH - (NBUF - PFD), NCH):
            pltpu.make_async_copy(bufs[j % NBUF], acc.at[dst_v.at[j]],
                                  ssems[j % NBUF]).wait()

        plsc.subcore_barrier()
        pltpu.sync_copy(acc.at[pl.ds(s * ROWS_PER_TILE, ROWS_PER_TILE)],
                        out_hbm.at[c, pl.ds(s * ROWS_PER_TILE, ROWS_PER_TILE)])

    return agg(table, src2d, dst2d, ew2d)


def _mm1_body(x_ref, w_ref, o_ref):
    o_ref[...] = jnp.dot(x_ref[...], w_ref[...],
                         preferred_element_type=jnp.float32)


def _norm_mm2_body(p_ref, w2_ref, o_ref):
    a = p_ref[0, :N] + p_ref[1, :N]
    nrm = jnp.sqrt(jnp.sum(a * a, axis=1, keepdims=True))
    h = jnp.maximum(a / jnp.maximum(nrm, 1e-12), 0.0)
    o_ref[...] = jnp.dot(h, w2_ref[...], preferred_element_type=jnp.float32)


def _loss_body(lp_ref, lab_ref, mask_ref, w1_ref, loss_ref, acc_ref):
    l = lp_ref[0, :N, 0:2] + lp_ref[1, :N, 0:2]
    l0 = l[:, 0:1]
    l1 = l[:, 1:2]
    mx = jnp.maximum(l0, l1)
    lse = mx + jnp.log(jnp.exp(l0 - mx) + jnp.exp(l1 - mx))
    lab = lab_ref[...]
    ly = jnp.where(lab == 1, l1, l0)
    ce = lse - ly
    mk = mask_ref[...]
    m = mk / jnp.mean(mk)
    wd = (WEIGHT_DECAY * 0.5) * jnp.sum(w1_ref[...] * w1_ref[...])
    loss_ref[...] = (wd + jnp.mean(ce * m)).reshape(1, 1)
    pred = (l1 > l0).astype(jnp.int32)
    corr = (pred == lab).astype(jnp.float32)
    acc_ref[...] = jnp.mean(corr * m).reshape(1, 1)


def kernel(x, edge_index, edge_weight, labels, mask, W1, W2):
    src3d = edge_index[0].reshape(NT, NCH, G)
    dst3d = edge_index[1].reshape(NT, NCH, G)
    ew3d = edge_weight.reshape(NT, NCH, G)
    w2p = jnp.zeros((H, CP), jnp.float32).at[:, :C].set(W2)

    pre1 = pl.pallas_call(
        _mm1_body,
        out_shape=jax.ShapeDtypeStruct((N, H), jnp.float32),
    )(x, W1)

    p1 = _sc_aggregate(pre1, src3d, dst3d, ew3d, H)

    pre2 = pl.pallas_call(
        _norm_mm2_body,
        out_shape=jax.ShapeDtypeStruct((N, CP), jnp.float32),
    )(p1, w2p)

    p2 = _sc_aggregate(pre2, src3d, dst3d, ew3d, CP)

    loss, acc = pl.pallas_call(
        _loss_body,
        out_shape=[jax.ShapeDtypeStruct((1, 1), jnp.float32),
                   jax.ShapeDtypeStruct((1, 1), jnp.float32)],
    )(p2, labels.reshape(N, 1), mask.reshape(N, 1), W1)

    return (loss[0, 0], acc[0, 0])


# speedup vs baseline: 11.9792x; 1.1104x over previous
import functools

import jax
import jax.numpy as jnp
from jax import lax
from jax.experimental import pallas as pl
from jax.experimental.pallas import tpu as pltpu
from jax.experimental.pallas import tpu_sc as plsc

N = 10000
E = 320000
D = 128
H = 64
C = 2
CP = 16
WEIGHT_DECAY = 0.0005

NC = 2
NS = 16
NT = NC * NS
G = 80
NCH = E // (NT * G)
NBUF = 3
PFD = 2
ROWS_PER_TILE = N // NS


def _sc_aggregate(table, src2d, dst2d, ew2d, F, stage_table):
    mesh = plsc.VectorSubcoreMesh(core_axis_name="c", subcore_axis_name="s")
    qf = F // 16

    @functools.partial(
        pl.kernel,
        out_type=jax.ShapeDtypeStruct((NC, N, F), jnp.float32),
        mesh=mesh,
        compiler_params=pltpu.CompilerParams(use_tc_tiling_on_sc=False),
        scratch_types=[
            pltpu.VMEM((NCH, G), jnp.int32),
            pltpu.VMEM((NCH, G), jnp.int32),
            pltpu.VMEM((NCH, G), jnp.float32),
            [pltpu.VMEM((G, F), jnp.float32)] * NBUF,
            pltpu.VMEM((ROWS_PER_TILE, F), jnp.float32),
            pltpu.VMEM_SHARED((N, F), jnp.float32),
            pltpu.VMEM_SHARED((N, F), jnp.float32) if stage_table else None,
            [pltpu.SemaphoreType.DMA] * NBUF,
            [pltpu.SemaphoreType.DMA] * NBUF,
        ],
    )
    def agg(table_hbm, src_hbm, dst_hbm, ew_hbm, out_hbm,
            src_v, dst_v, ew_v, bufs, zbuf, acc, tbl, gsems, ssems):
        c = lax.axis_index("c")
        s = lax.axis_index("s")
        tid = c * NS + s

        pltpu.sync_copy(src_hbm.at[tid], src_v)
        pltpu.sync_copy(dst_hbm.at[tid], dst_v)
        pltpu.sync_copy(ew_hbm.at[tid], ew_v)
        if stage_table:
            pltpu.sync_copy(
                table_hbm.at[pl.ds(s * ROWS_PER_TILE, ROWS_PER_TILE)],
                tbl.at[pl.ds(s * ROWS_PER_TILE, ROWS_PER_TILE)])
        gsrc = tbl if stage_table else table_hbm

        def zrow(r, carry):
            for q in range(qf):
                zbuf[r, pl.ds(q * 16, 16)] = jnp.zeros((16,), jnp.float32)
            return carry
        lax.fori_loop(0, ROWS_PER_TILE, zrow, 0)
        pltpu.sync_copy(zbuf, acc.at[pl.ds(s * ROWS_PER_TILE, ROWS_PER_TILE)])
        plsc.subcore_barrier()

        def gather(jj, b):
            pltpu.async_copy(gsrc.at[src_v.at[jj]], bufs[b], gsems[b])

        def scatter(jj, b):
            pltpu.async_copy(bufs[b], acc.at[dst_v.at[jj]], ssems[b], add=True)

        for j in range(PFD):
            gather(j, j % NBUF)

        def step(jj, b):
            pltpu.make_async_copy(gsrc.at[src_v.at[jj]], bufs[b],
                                  gsems[b]).wait()

            def mul16(t, carry2):
                wv = ew_v[jj, pl.ds(t * 16, 16)]
                for u in range(16):
                    w = wv[u]
                    for q in range(qf):
                        sl = pl.ds(q * 16, 16)
                        bufs[b][t * 16 + u, sl] = bufs[b][t * 16 + u, sl] * w
                return carry2
            lax.fori_loop(0, G // 16, mul16, 0, unroll=False)

            scatter(jj, b)
            bn = (b + PFD) % NBUF

            @pl.when(jj >= NBUF - PFD)
            def _():
                pltpu.make_async_copy(bufs[bn], acc.at[dst_v.at[jj]],
                                      ssems[bn]).wait()

            @pl.when(jj + PFD < NCH)
            def _():
                gather(jj + PFD, bn)

        def block(it, carry):
            for b in range(NBUF):
                step(it * NBUF + b, b)
            return carry
        lax.fori_loop(0, NCH // NBUF, block, 0)
        for j in range(NCH - NCH % NBUF, NCH):
            step(jnp.int32(j), j % NBUF)

        for j in range(NCH - (NBUF - PFD), NCH):
            pltpu.make_async_copy(bufs[j % NBUF], acc.at[dst_v.at[j]],
                                  ssems[j % NBUF]).wait()

        plsc.subcore_barrier()
        pltpu.sync_copy(acc.at[pl.ds(s * ROWS_PER_TILE, ROWS_PER_TILE)],
                        out_hbm.at[c, pl.ds(s * ROWS_PER_TILE, ROWS_PER_TILE)])

    return agg(table, src2d, dst2d, ew2d)


def _mm1_body(x_ref, w_ref, o_ref):
    o_ref[...] = jnp.dot(x_ref[...], w_ref[...],
                         preferred_element_type=jnp.float32)


def _norm_mm2_body(p_ref, w2_ref, o_ref):
    a = p_ref[0, :N] + p_ref[1, :N]
    nrm = jnp.sqrt(jnp.sum(a * a, axis=1, keepdims=True))
    h = jnp.maximum(a / jnp.maximum(nrm, 1e-12), 0.0)
    o_ref[...] = jnp.dot(h, w2_ref[...], preferred_element_type=jnp.float32)


def _loss_body(lp_ref, lab_ref, mask_ref, w1_ref, loss_ref, acc_ref):
    l = lp_ref[0, :N, 0:2] + lp_ref[1, :N, 0:2]
    l0 = l[:, 0:1]
    l1 = l[:, 1:2]
    mx = jnp.maximum(l0, l1)
    lse = mx + jnp.log(jnp.exp(l0 - mx) + jnp.exp(l1 - mx))
    lab = lab_ref[...]
    ly = jnp.where(lab == 1, l1, l0)
    ce = lse - ly
    mk = mask_ref[...]
    m = mk / jnp.mean(mk)
    wd = (WEIGHT_DECAY * 0.5) * jnp.sum(w1_ref[...] * w1_ref[...])
    loss_ref[...] = (wd + jnp.mean(ce * m)).reshape(1, 1)
    pred = (l1 > l0).astype(jnp.int32)
    corr = (pred == lab).astype(jnp.float32)
    acc_ref[...] = jnp.mean(corr * m).reshape(1, 1)


def kernel(x, edge_index, edge_weight, labels, mask, W1, W2):
    src3d = edge_index[0].reshape(NT, NCH, G)
    dst3d = edge_index[1].reshape(NT, NCH, G)
    ew3d = edge_weight.reshape(NT, NCH, G)
    w2p = jnp.zeros((H, CP), jnp.float32).at[:, :C].set(W2)

    pre1 = pl.pallas_call(
        _mm1_body,
        out_shape=jax.ShapeDtypeStruct((N, H), jnp.float32),
    )(x, W1)

    p1 = _sc_aggregate(pre1, src3d, dst3d, ew3d, H, stage_table=False)

    pre2 = pl.pallas_call(
        _norm_mm2_body,
        out_shape=jax.ShapeDtypeStruct((N, CP), jnp.float32),
    )(p1, w2p)

    p2 = _sc_aggregate(pre2, src3d, dst3d, ew3d, CP, stage_table=True)

    loss, acc = pl.pallas_call(
        _loss_body,
        out_shape=[jax.ShapeDtypeStruct((1, 1), jnp.float32),
                   jax.ShapeDtypeStruct((1, 1), jnp.float32)],
    )(p2, labels.reshape(N, 1), mask.reshape(N, 1), W1)

    return (loss[0, 0], acc[0, 0])


# speedup vs baseline: 16.1152x; 1.3453x over previous
import functools

import jax
import jax.numpy as jnp
from jax import lax
from jax.experimental import pallas as pl
from jax.experimental.pallas import tpu as pltpu
from jax.experimental.pallas import tpu_sc as plsc

N = 10000
E = 320000
D = 128
H = 64
HH = 32
C = 2
CP = 16
WEIGHT_DECAY = 0.0005

NC = 2
NS = 16
NT = NC * NS
NBUF = 3
PFD = 2
ROWS_PER_TILE = N // NS

EPT1 = E // NS
G1 = 200
SUBD = 2
GS = G1 // SUBD
NCH1 = EPT1 // G1

G2 = 80
NCH2 = E // (NT * G2)


def _zero_acc_stripe(bufs, acc, base, G, F, qf):
    def zrow(r, carry):
        for q in range(qf):
            bufs[0][r, pl.ds(q * 16, 16)] = jnp.zeros((16,), jnp.float32)
        return carry
    lax.fori_loop(0, G, zrow, 0)
    done = 0
    while done < ROWS_PER_TILE:
        step_rows = min(G, ROWS_PER_TILE - done)
        pltpu.sync_copy(bufs[0].at[pl.ds(0, step_rows)],
                        acc.at[pl.ds(base + done, step_rows)])
        done += step_rows


def _mul_weights(bufs, b, ew_v, jj, G, qf):
    def mul16(t, carry):
        wv = ew_v[jj, pl.ds(t * 16, 16)]
        for u in range(16):
            w = wv[u]
            for q in range(qf):
                sl = pl.ds(q * 16, 16)
                bufs[b][t * 16 + u, sl] = bufs[b][t * 16 + u, sl] * w
        return carry
    lax.fori_loop(0, G // 16, mul16, 0, unroll=False)


def _sc_layer1(table2, src4, dst4, ew3):
    mesh = plsc.VectorSubcoreMesh(core_axis_name="c", subcore_axis_name="s")

    @functools.partial(
        pl.kernel,
        out_type=jax.ShapeDtypeStruct((NC, N, HH), jnp.float32),
        mesh=mesh,
        name="sc_agg_l1",
        compiler_params=pltpu.CompilerParams(use_tc_tiling_on_sc=False),
        scratch_types=[
            pltpu.VMEM((NCH1, SUBD, GS), jnp.int32),
            pltpu.VMEM((NCH1, SUBD, GS), jnp.int32),
            pltpu.VMEM((NCH1, G1), jnp.float32),
            [pltpu.VMEM((G1, HH), jnp.float32)] * NBUF,
            pltpu.VMEM_SHARED((N, HH), jnp.float32),
            pltpu.VMEM_SHARED((N, HH), jnp.float32),
            [pltpu.SemaphoreType.DMA] * NBUF,
            [pltpu.SemaphoreType.DMA] * NBUF,
        ],
    )
    def agg(table_hbm, src_hbm, dst_hbm, ew_hbm, out_hbm,
            src_v, dst_v, ew_v, bufs, acc, tbl, gsems, ssems):
        c = lax.axis_index("c")
        s = lax.axis_index("s")
        base = s * ROWS_PER_TILE

        pltpu.sync_copy(src_hbm.at[s], src_v)
        pltpu.sync_copy(dst_hbm.at[s], dst_v)
        pltpu.sync_copy(ew_hbm.at[s], ew_v)
        pltpu.sync_copy(table_hbm.at[c, pl.ds(base, ROWS_PER_TILE)],
                        tbl.at[pl.ds(base, ROWS_PER_TILE)])

        _zero_acc_stripe(bufs, acc, base, G1, HH, HH // 16)
        plsc.subcore_barrier()

        def gather(jj, b):
            for k in range(SUBD):
                pltpu.async_copy(tbl.at[src_v.at[jj, k]],
                                 bufs[b].at[pl.ds(k * GS, GS)], gsems[b])

        def wait_gathers(b):
            for k in range(SUBD):
                pltpu.make_async_copy(tbl.at[src_v.at[0, k]],
                                      bufs[b].at[pl.ds(k * GS, GS)],
                                      gsems[b]).wait()

        def wait_scatters(b):
            for k in range(SUBD):
                pltpu.make_async_copy(bufs[b].at[pl.ds(k * GS, GS)],
                                      acc.at[dst_v.at[0, k]],
                                      ssems[b]).wait()

        def scatter(jj, b):
            for k in range(SUBD):
                pltpu.async_copy(bufs[b].at[pl.ds(k * GS, GS)],
                                 acc.at[dst_v.at[jj, k]], ssems[b], add=True)

        for j in range(PFD):
            gather(j, j % NBUF)

        def step(jj, b):
            wait_gathers(b)
            _mul_weights(bufs, b, ew_v, jj, G1, HH // 16)
            scatter(jj, b)
            bn = (b + PFD) % NBUF

            @pl.when(jj >= NBUF - PFD)
            def _():
                wait_scatters(bn)

            @pl.when(jj + PFD < NCH1)
            def _():
                gather(jj + PFD, bn)

        def block(it, carry):
            for b in range(NBUF):
                step(it * NBUF + b, b)
            return carry
        lax.fori_loop(0, NCH1 // NBUF, block, 0)
        for j in range(NCH1 - NCH1 % NBUF, NCH1):
            step(jnp.int32(j), j % NBUF)
        for j in range(NCH1 - (NBUF - PFD), NCH1):
            wait_scatters(j % NBUF)

        plsc.subcore_barrier()
        pltpu.sync_copy(acc.at[pl.ds(base, ROWS_PER_TILE)],
                        out_hbm.at[c, pl.ds(base, ROWS_PER_TILE)])

    return agg(table2, src4, dst4, ew3)


def _sc_layer2(table, src3, dst3, ew3):
    mesh = plsc.VectorSubcoreMesh(core_axis_name="c", subcore_axis_name="s")

    @functools.partial(
        pl.kernel,
        out_type=jax.ShapeDtypeStruct((NC, N, CP), jnp.float32),
        mesh=mesh,
        name="sc_agg_l2",
        compiler_params=pltpu.CompilerParams(use_tc_tiling_on_sc=False),
        scratch_types=[
            pltpu.VMEM((NCH2, G2), jnp.int32),
            pltpu.VMEM((NCH2, G2), jnp.int32),
            pltpu.VMEM((NCH2, G2), jnp.float32),
            [pltpu.VMEM((G2, CP), jnp.float32)] * NBUF,
            pltpu.VMEM_SHARED((N, CP), jnp.float32),
            pltpu.VMEM_SHARED((N, CP), jnp.float32),
            [pltpu.SemaphoreType.DMA] * NBUF,
            [pltpu.SemaphoreType.DMA] * NBUF,
        ],
    )
    def agg(table_hbm, src_hbm, dst_hbm, ew_hbm, out_hbm,
            src_v, dst_v, ew_v, bufs, acc, tbl, gsems, ssems):
        c = lax.axis_index("c")
        s = lax.axis_index("s")
        tid = c * NS + s
        base = s * ROWS_PER_TILE

        pltpu.sync_copy(src_hbm.at[tid], src_v)
        pltpu.sync_copy(dst_hbm.at[tid], dst_v)
        pltpu.sync_copy(ew_hbm.at[tid], ew_v)
        pltpu.sync_copy(table_hbm.at[pl.ds(base, ROWS_PER_TILE)],
                        tbl.at[pl.ds(base, ROWS_PER_TILE)])

        _zero_acc_stripe(bufs, acc, base, G2, CP, CP // 16)
        plsc.subcore_barrier()

        def gather(jj, b):
            pltpu.async_copy(tbl.at[src_v.at[jj]], bufs[b], gsems[b])

        def scatter(jj, b):
            pltpu.async_copy(bufs[b], acc.at[dst_v.at[jj]], ssems[b], add=True)

        for j in range(PFD):
            gather(j, j % NBUF)

        def step(jj, b):
            pltpu.make_async_copy(tbl.at[src_v.at[jj]], bufs[b],
                                  gsems[b]).wait()
            _mul_weights(bufs, b, ew_v, jj, G2, CP // 16)
            scatter(jj, b)
            bn = (b + PFD) % NBUF

            @pl.when(jj >= NBUF - PFD)
            def _():
                pltpu.make_async_copy(bufs[bn], acc.at[dst_v.at[jj]],
                                      ssems[bn]).wait()

            @pl.when(jj + PFD < NCH2)
            def _():
                gather(jj + PFD, bn)

        def block(it, carry):
            for b in range(NBUF):
                step(it * NBUF + b, b)
            return carry
        lax.fori_loop(0, NCH2 // NBUF, block, 0)
        for j in range(NCH2 - NCH2 % NBUF, NCH2):
            step(jnp.int32(j), j % NBUF)
        for j in range(NCH2 - (NBUF - PFD), NCH2):
            pltpu.make_async_copy(bufs[j % NBUF], acc.at[dst_v.at[j]],
                                  ssems[j % NBUF]).wait()

        plsc.subcore_barrier()
        pltpu.sync_copy(acc.at[pl.ds(base, ROWS_PER_TILE)],
                        out_hbm.at[c, pl.ds(base, ROWS_PER_TILE)])

    return agg(table, src3, dst3, ew3)


def _mm1_body(x_ref, w_ref, o_ref):
    xv = x_ref[...]
    o_ref[0] = jnp.dot(xv, w_ref[:, :HH], preferred_element_type=jnp.float32)
    o_ref[1] = jnp.dot(xv, w_ref[:, HH:], preferred_element_type=jnp.float32)


def _norm_mm2_body(p_ref, w2_ref, o_ref):
    a = jnp.concatenate([p_ref[0, :N], p_ref[1, :N]], axis=1)
    nrm = jnp.sqrt(jnp.sum(a * a, axis=1, keepdims=True))
    h = jnp.maximum(a / jnp.maximum(nrm, 1e-12), 0.0)
    o_ref[...] = jnp.dot(h, w2_ref[...], preferred_element_type=jnp.float32)


def _loss_body(lp_ref, lab_ref, mask_ref, w1_ref, loss_ref, acc_ref):
    l = lp_ref[0, :N, 0:2] + lp_ref[1, :N, 0:2]
    l0 = l[:, 0:1]
    l1 = l[:, 1:2]
    mx = jnp.maximum(l0, l1)
    lse = mx + jnp.log(jnp.exp(l0 - mx) + jnp.exp(l1 - mx))
    lab = lab_ref[...]
    ly = jnp.where(lab == 1, l1, l0)
    ce = lse - ly
    mk = mask_ref[...]
    m = mk / jnp.mean(mk)
    wd = (WEIGHT_DECAY * 0.5) * jnp.sum(w1_ref[...] * w1_ref[...])
    loss_ref[...] = (wd + jnp.mean(ce * m)).reshape(1, 1)
    pred = (l1 > l0).astype(jnp.int32)
    corr = (pred == lab).astype(jnp.float32)
    acc_ref[...] = jnp.mean(corr * m).reshape(1, 1)


def kernel(x, edge_index, edge_weight, labels, mask, W1, W2):
    src, dst = edge_index[0], edge_index[1]
    w2p = jnp.zeros((H, CP), jnp.float32).at[:, :C].set(W2)

    pre1 = pl.pallas_call(
        _mm1_body,
        out_shape=jax.ShapeDtypeStruct((NC, N, HH), jnp.float32),
    )(x, W1)

    p1 = _sc_layer1(pre1,
                    src.reshape(NS, NCH1, SUBD, GS),
                    dst.reshape(NS, NCH1, SUBD, GS),
                    edge_weight.reshape(NS, NCH1, G1))

    pre2 = pl.pallas_call(
        _norm_mm2_body,
        out_shape=jax.ShapeDtypeStruct((N, CP), jnp.float32),
    )(p1, w2p)

    p2 = _sc_layer2(pre2,
                    src.reshape(NT, NCH2, G2),
                    dst.reshape(NT, NCH2, G2),
                    edge_weight.reshape(NT, NCH2, G2))

    loss, acc = pl.pallas_call(
        _loss_body,
        out_shape=[jax.ShapeDtypeStruct((1, 1), jnp.float32),
                   jax.ShapeDtypeStruct((1, 1), jnp.float32)],
    )(p2, labels.reshape(N, 1), mask.reshape(N, 1), W1)

    return (loss[0, 0], acc[0, 0])
